# Initial kernel scaffold; baseline (speedup 1.0000x reference)
import jax
import jax.numpy as jnp
from jax.experimental import pallas as pl


def kernel(features, edge_index, W1a, b1a, W1b, b1b, W2a, b2a, W2b, b2b, W3a, b3a, W3b, b3b):
    raise NotImplementedError("write your pallas kernel here")


# speedup vs baseline: 5.4786x; 5.4786x over previous
import functools

import jax
import jax.numpy as jnp
from jax import lax
from jax.experimental import pallas as pl
from jax.experimental.pallas import tpu as pltpu
from jax.experimental.pallas import tpu_sc as plsc

N = 10000
E = 320000
D = 128

NC = 2
NS = 16
NW = NC * NS
E_PER_W = E // NW
CHUNK = 80
N_CHUNKS = E_PER_W // CHUNK
RPT = 624
TAIL = N - NS * RPT


def _make_sc_agg(compute_deg: bool):
  mesh = plsc.VectorSubcoreMesh(core_axis_name="c", subcore_axis_name="s")
  out_type = [jax.ShapeDtypeStruct((NC, N, D), jnp.float32)]
  if compute_deg:
    out_type.append(jax.ShapeDtypeStruct((NC, N), jnp.float32))

  @functools.partial(
      pl.kernel,
      mesh=mesh,
      out_type=tuple(out_type),
      scratch_types=(
          pltpu.VMEM((CHUNK,), jnp.int32),
          pltpu.VMEM((CHUNK,), jnp.int32),
          pltpu.VMEM((CHUNK, D), jnp.float32),
          pltpu.VMEM((CHUNK,), jnp.float32),
          pltpu.VMEM_SHARED((N, D), jnp.float32),
          pltpu.VMEM_SHARED((N,), jnp.float32),
          pltpu.SemaphoreType.DMA,
      ),
  )
  def sc_agg(x_hbm, src_hbm, dst_hbm, zrow_hbm, zdeg_hbm, *rest):
    if compute_deg:
      agg_out, deg_out = rest[0], rest[1]
      scratch = rest[2:]
    else:
      agg_out = rest[0]
      scratch = rest[1:]
    src_v, dst_v, rows_v, ones_v, acc, deg_acc, sem = scratch

    c = lax.axis_index("c")
    s = lax.axis_index("s")
    wid = c * NS + s

    pltpu.sync_copy(zrow_hbm.at[pl.ds(s * RPT, RPT)],
                    acc.at[pl.ds(s * RPT, RPT)])
    @pl.when(s == NS - 1)
    def _():
      pltpu.sync_copy(zrow_hbm.at[pl.ds(NS * RPT, TAIL)],
                      acc.at[pl.ds(NS * RPT, TAIL)])
    if compute_deg:
      @pl.when(s == 0)
      def _():
        pltpu.sync_copy(zdeg_hbm, deg_acc)
      for i in range(CHUNK // 16):
        ones_v[pl.ds(i * 16, 16)] = jnp.ones((16,), jnp.float32)
    plsc.subcore_barrier()

    def body(i, carry):
      base = wid * E_PER_W + i * CHUNK
      pltpu.sync_copy(src_hbm.at[pl.ds(base, CHUNK)], src_v)
      pltpu.sync_copy(dst_hbm.at[pl.ds(base, CHUNK)], dst_v)
      pltpu.async_copy(x_hbm.at[src_v], rows_v, sem).wait()
      pltpu.sync_copy(rows_v, acc.at[dst_v], add=True)
      if compute_deg:
        pltpu.sync_copy(ones_v, deg_acc.at[dst_v], add=True)
      return carry

    lax.fori_loop(0, N_CHUNKS, body, 0)
    plsc.subcore_barrier()

    pltpu.sync_copy(acc.at[pl.ds(s * RPT, RPT)],
                    agg_out.at[c].at[pl.ds(s * RPT, RPT)])
    @pl.when(s == NS - 1)
    def _():
      pltpu.sync_copy(acc.at[pl.ds(NS * RPT, TAIL)],
                      agg_out.at[c].at[pl.ds(NS * RPT, TAIL)])
    if compute_deg:
      @pl.when(s == 0)
      def _():
        pltpu.sync_copy(deg_acc, deg_out.at[c])

  return sc_agg


_sc_agg_deg = _make_sc_agg(True)
_sc_agg = _make_sc_agg(False)


BLK = 1000


def _tc_layer1(x_ref, agg_ref, deg_ref, wa_ref, ba_ref, wb_ref, bb_ref, o_ref):
  deg = deg_ref[0] + deg_ref[1]
  agg = agg_ref[0] + agg_ref[1]
  h = x_ref[...] + agg * (1.0 / jnp.maximum(deg, 1.0))
  t = jnp.maximum(jnp.dot(h, wa_ref[...], preferred_element_type=jnp.float32)
                  + ba_ref[...], 0.0)
  y = jnp.dot(t, wb_ref[...], preferred_element_type=jnp.float32) + bb_ref[...]
  o_ref[...] = jnp.maximum(y, 0.0)


def _tc_layer23(x_ref, agg_ref, deg_ref, w2a_ref, b2a_ref, w2b_ref, b2b_ref,
                w3a_ref, b3a_ref, w3b_ref, b3b_ref, o_ref):
  deg = deg_ref[0] + deg_ref[1]
  agg = agg_ref[0] + agg_ref[1]
  h = x_ref[...] + agg * (1.0 / jnp.maximum(deg, 1.0))
  t = jnp.maximum(jnp.dot(h, w2a_ref[...], preferred_element_type=jnp.float32)
                  + b2a_ref[...], 0.0)
  x2 = jnp.maximum(jnp.dot(t, w2b_ref[...], preferred_element_type=jnp.float32)
                   + b2b_ref[...], 0.0)
  t3 = jnp.maximum(jnp.dot(x2, w3a_ref[...], preferred_element_type=jnp.float32)
                   + b3a_ref[...], 0.0)
  o_ref[...] = (jnp.dot(t3, w3b_ref[...], preferred_element_type=jnp.float32)
                + b3b_ref[...])


def _row_spec():
  return pl.BlockSpec((BLK, D), lambda i: (i, 0))


def _agg_spec():
  return pl.BlockSpec((NC, BLK, D), lambda i: (0, i, 0))


def _deg_spec():
  return pl.BlockSpec((NC, BLK, 1), lambda i: (0, i, 0))


def _w_spec():
  return pl.BlockSpec((D, D), lambda i: (0, 0))


def _b_spec():
  return pl.BlockSpec((1, D), lambda i: (0, 0))


def kernel(features, edge_index, W1a, b1a, W1b, b1b, W2a, b2a, W2b, b2b,
           W3a, b3a, W3b, b3b):
  src = edge_index[0]
  dst = edge_index[1]
  zrow = jnp.zeros((N, D), jnp.float32)
  zdeg = jnp.zeros((N,), jnp.float32)

  aggp1, degp = _sc_agg_deg(features, src, dst, zrow, zdeg)
  degp3 = degp.reshape(NC, N, 1)

  x1 = pl.pallas_call(
      _tc_layer1,
      grid=(N // BLK,),
      in_specs=[_row_spec(), _agg_spec(), _deg_spec(),
                _w_spec(), _b_spec(), _w_spec(), _b_spec()],
      out_specs=_row_spec(),
      out_shape=jax.ShapeDtypeStruct((N, D), jnp.float32),
  )(features, aggp1, degp3, W1a, b1a.reshape(1, D), W1b, b1b.reshape(1, D))

  (aggp2,) = _sc_agg(x1, src, dst, zrow, zdeg)

  out = pl.pallas_call(
      _tc_layer23,
      grid=(N // BLK,),
      in_specs=[_row_spec(), _agg_spec(), _deg_spec(),
                _w_spec(), _b_spec(), _w_spec(), _b_spec(),
                _w_spec(), _b_spec(), _w_spec(), _b_spec()],
      out_specs=_row_spec(),
      out_shape=jax.ShapeDtypeStruct((N, D), jnp.float32),
  )(x1, aggp2, degp3, W2a, b2a.reshape(1, D), W2b, b2b.reshape(1, D),
    W3a, b3a.reshape(1, D), W3b, b3b.reshape(1, D))
  return out


# speedup vs baseline: 12.0159x; 2.1932x over previous
import functools

import jax
import jax.numpy as jnp
from jax import lax
from jax.experimental import pallas as pl
from jax.experimental.pallas import tpu as pltpu
from jax.experimental.pallas import tpu_sc as plsc

N = 10000
E = 320000
D = 128

NC = 2
NS = 16
NW = NC * NS
E_PER_W = E // NW
CHUNK = 100
N_CHUNKS = E_PER_W // CHUNK
KBLK = 20
NBLK = N_CHUNKS // KBLK
RPT = 624
TAIL = N - NS * RPT


def _make_sc_agg(compute_deg: bool):
  mesh = plsc.VectorSubcoreMesh(core_axis_name="c", subcore_axis_name="s")
  out_type = [jax.ShapeDtypeStruct((NC, N, D), jnp.float32)]
  if compute_deg:
    out_type.append(jax.ShapeDtypeStruct((NC, N), jnp.float32))

  @functools.partial(
      pl.kernel,
      mesh=mesh,
      out_type=tuple(out_type),
      scratch_types=(
          pltpu.VMEM((KBLK, CHUNK), jnp.int32),
          pltpu.VMEM((KBLK, CHUNK), jnp.int32),
          pltpu.VMEM((CHUNK, D), jnp.float32),
          pltpu.VMEM((CHUNK, D), jnp.float32),
          pltpu.VMEM((CHUNK,), jnp.float32),
          pltpu.VMEM_SHARED((N, D), jnp.float32),
          pltpu.VMEM_SHARED((N,), jnp.float32),
          pltpu.SemaphoreType.DMA,
          pltpu.SemaphoreType.DMA,
          pltpu.SemaphoreType.DMA,
      ),
  )
  def sc_agg(x_hbm, src_hbm, dst_hbm, zrow_hbm, zdeg_hbm, *rest):
    if compute_deg:
      agg_out, deg_out = rest[0], rest[1]
      scratch = rest[2:]
    else:
      agg_out = rest[0]
      scratch = rest[1:]
    src_v, dst_v, rows0, rows1, ones_v, acc, deg_acc, g0, g1, dsem = scratch
    rows = (rows0, rows1)
    gsem = (g0, g1)

    c = lax.axis_index("c")
    s = lax.axis_index("s")
    wid = c * NS + s

    pltpu.sync_copy(zrow_hbm.at[pl.ds(s * RPT, RPT)],
                    acc.at[pl.ds(s * RPT, RPT)])
    @pl.when(s == NS - 1)
    def _():
      pltpu.sync_copy(zrow_hbm.at[pl.ds(NS * RPT, TAIL)],
                      acc.at[pl.ds(NS * RPT, TAIL)])
    if compute_deg:
      @pl.when(s == 0)
      def _():
        pltpu.sync_copy(zdeg_hbm, deg_acc)
      for i in range(CHUNK // 20):
        ones_v[pl.ds(i * 20, 20)] = jnp.ones((20,), jnp.float32)
    plsc.subcore_barrier()

    def blk_body(j, carry):
      pltpu.sync_copy(src_hbm.at[wid].at[j], src_v)
      pltpu.sync_copy(dst_hbm.at[wid].at[j], dst_v)
      pltpu.async_copy(x_hbm.at[src_v.at[0]], rows0, g0)
      pltpu.async_copy(x_hbm.at[src_v.at[1]], rows1, g1)

      def body(o, carry2):
        for b in range(2):
          i = o * 2 + b
          pltpu.make_async_copy(x_hbm.at[src_v.at[i]], rows[b], gsem[b]).wait()
          if compute_deg:
            pltpu.async_copy(ones_v, deg_acc.at[dst_v.at[i]], dsem, add=True)
          pltpu.sync_copy(rows[b], acc.at[dst_v.at[i]], add=True)
          @pl.when(i + 2 < KBLK)
          def _():
            pltpu.async_copy(x_hbm.at[src_v.at[i + 2]], rows[b], gsem[b])
        return carry2

      lax.fori_loop(0, KBLK // 2, body, 0)
      return carry

    lax.fori_loop(0, NBLK, blk_body, 0)
    if compute_deg:
      def drain(i, carry):
        pltpu.make_async_copy(ones_v, deg_acc.at[dst_v.at[0]], dsem).wait()
        return carry
      lax.fori_loop(0, N_CHUNKS, drain, 0)
    plsc.subcore_barrier()

    pltpu.sync_copy(acc.at[pl.ds(s * RPT, RPT)],
                    agg_out.at[c].at[pl.ds(s * RPT, RPT)])
    @pl.when(s == NS - 1)
    def _():
      pltpu.sync_copy(acc.at[pl.ds(NS * RPT, TAIL)],
                      agg_out.at[c].at[pl.ds(NS * RPT, TAIL)])
    if compute_deg:
      @pl.when(s == 0)
      def _():
        pltpu.sync_copy(deg_acc, deg_out.at[c])

  return sc_agg


_sc_agg_deg = _make_sc_agg(True)
_sc_agg = _make_sc_agg(False)


BLK = 1000


def _tc_layer1(x_ref, agg_ref, deg_ref, wa_ref, ba_ref, wb_ref, bb_ref, o_ref):
  deg = deg_ref[0] + deg_ref[1]
  agg = agg_ref[0] + agg_ref[1]
  h = x_ref[...] + agg * (1.0 / jnp.maximum(deg, 1.0))
  t = jnp.maximum(jnp.dot(h, wa_ref[...], preferred_element_type=jnp.float32)
                  + ba_ref[...], 0.0)
  y = jnp.dot(t, wb_ref[...], preferred_element_type=jnp.float32) + bb_ref[...]
  o_ref[...] = jnp.maximum(y, 0.0)


def _tc_layer23(x_ref, agg_ref, deg_ref, w2a_ref, b2a_ref, w2b_ref, b2b_ref,
                w3a_ref, b3a_ref, w3b_ref, b3b_ref, o_ref):
  deg = deg_ref[0] + deg_ref[1]
  agg = agg_ref[0] + agg_ref[1]
  h = x_ref[...] + agg * (1.0 / jnp.maximum(deg, 1.0))
  t = jnp.maximum(jnp.dot(h, w2a_ref[...], preferred_element_type=jnp.float32)
                  + b2a_ref[...], 0.0)
  x2 = jnp.maximum(jnp.dot(t, w2b_ref[...], preferred_element_type=jnp.float32)
                   + b2b_ref[...], 0.0)
  t3 = jnp.maximum(jnp.dot(x2, w3a_ref[...], preferred_element_type=jnp.float32)
                   + b3a_ref[...], 0.0)
  o_ref[...] = (jnp.dot(t3, w3b_ref[...], preferred_element_type=jnp.float32)
                + b3b_ref[...])


def _row_spec():
  return pl.BlockSpec((BLK, D), lambda i: (i, 0))


def _agg_spec():
  return pl.BlockSpec((NC, BLK, D), lambda i: (0, i, 0))


def _deg_spec():
  return pl.BlockSpec((NC, BLK, 1), lambda i: (0, i, 0))


def _w_spec():
  return pl.BlockSpec((D, D), lambda i: (0, 0))


def _b_spec():
  return pl.BlockSpec((1, D), lambda i: (0, 0))


def kernel(features, edge_index, W1a, b1a, W1b, b1b, W2a, b2a, W2b, b2b,
           W3a, b3a, W3b, b3b):
  src = edge_index[0].reshape(NW, NBLK, KBLK, CHUNK)
  dst = edge_index[1].reshape(NW, NBLK, KBLK, CHUNK)
  zrow = jnp.zeros((N, D), jnp.float32)
  zdeg = jnp.zeros((N,), jnp.float32)

  aggp1, degp = _sc_agg_deg(features, src, dst, zrow, zdeg)
  degp3 = degp.reshape(NC, N, 1)

  x1 = pl.pallas_call(
      _tc_layer1,
      grid=(N // BLK,),
      in_specs=[_row_spec(), _agg_spec(), _deg_spec(),
                _w_spec(), _b_spec(), _w_spec(), _b_spec()],
      out_specs=_row_spec(),
      out_shape=jax.ShapeDtypeStruct((N, D), jnp.float32),
  )(features, aggp1, degp3, W1a, b1a.reshape(1, D), W1b, b1b.reshape(1, D))

  (aggp2,) = _sc_agg(x1, src, dst, zrow, zdeg)

  out = pl.pallas_call(
      _tc_layer23,
      grid=(N // BLK,),
      in_specs=[_row_spec(), _agg_spec(), _deg_spec(),
                _w_spec(), _b_spec(), _w_spec(), _b_spec(),
                _w_spec(), _b_spec(), _w_spec(), _b_spec()],
      out_specs=_row_spec(),
      out_shape=jax.ShapeDtypeStruct((N, D), jnp.float32),
  )(x1, aggp2, degp3, W2a, b2a.reshape(1, D), W2b, b2b.reshape(1, D),
    W3a, b3a.reshape(1, D), W3b, b3b.reshape(1, D))
  return out
